# baseline (speedup 1.0000x reference)
import jax
import jax.numpy as jnp
from jax import lax
from jax.experimental import pallas as pl
from jax.experimental.pallas import tpu as pltpu
from jax.experimental.pallas import tpu_sc as plsc

N_COINS = 100000
EMBED_DIM = 16
BATCH = 16384
HIST = 50
B_TOTAL = BATCH * HIST

NC = 2
NS = 16
NS_ACTIVE = 8
NW = NC * NS_ACTIVE
B_PER_W = B_TOTAL // NW
CHUNK = 1600
N_CHUNKS = B_PER_W // CHUNK


def _emb_body(idx_hbm, table_hbm, out_hbm, idx_v, rows0, rows1,
              gsem0, gsem1, ssem0, ssem1):
    sid = lax.axis_index("s")
    wid = sid * NC + lax.axis_index("c")
    rows = (rows0, rows1)
    gsems = (gsem0, gsem1)
    ssems = (ssem0, ssem1)

    @pl.when(sid < NS_ACTIVE)
    def _():
        base = wid * B_PER_W
        pltpu.sync_copy(idx_hbm.at[pl.ds(base, B_PER_W)], idx_v)
        sds = [None, None]
        for i in range(N_CHUNKS):
            b = i % 2
            if sds[b] is not None:
                sds[b].wait()
            pltpu.async_copy(
                table_hbm.at[idx_v.at[pl.ds(i * CHUNK, CHUNK)]], rows[b],
                gsems[b]).wait()
            sds[b] = pltpu.async_copy(
                rows[b], out_hbm.at[pl.ds(base + i * CHUNK, CHUNK)], ssems[b])
        for s in sds:
            s.wait()


def kernel(coin_id, table):
    idx = coin_id.reshape(-1).astype(jnp.int32)
    mesh = plsc.VectorSubcoreMesh(core_axis_name="c", subcore_axis_name="s")
    k = pl.kernel(
        _emb_body,
        mesh=mesh,
        out_type=jax.ShapeDtypeStruct((B_TOTAL, EMBED_DIM), jnp.float32),
        scratch_types=(
            [pltpu.VMEM((B_PER_W,), jnp.int32)]
            + [pltpu.VMEM((CHUNK, EMBED_DIM), jnp.float32)] * 2
            + [pltpu.SemaphoreType.DMA] * 4
        ),
        compiler_params=pltpu.CompilerParams(use_tc_tiling_on_sc=False),
    )
    out = k(idx, table)
    return out.reshape(BATCH, HIST, EMBED_DIM)


# speedup vs baseline: 1.4068x; 1.4068x over previous
import jax
import jax.numpy as jnp
from jax import lax
from jax.experimental import pallas as pl
from jax.experimental.pallas import tpu as pltpu
from jax.experimental.pallas import tpu_sc as plsc

N_COINS = 100000
EMBED_DIM = 16
BATCH = 16384
HIST = 50

NC = 2
NS = 16
NW = NC * NS
ROWS_PER_W = BATCH // NW
RCHUNK = 16
N_CHUNKS = ROWS_PER_W // RCHUNK


def _emb_body(idx_hbm, table_hbm, out_hbm, idx_v, rows_v, gsem):
    wid = lax.axis_index("s") * NC + lax.axis_index("c")
    base = wid * ROWS_PER_W

    def body(j, carry):
        r0 = base + j * RCHUNK
        pltpu.sync_copy(idx_hbm.at[pl.ds(r0, RCHUNK)], idx_v)
        gds = [
            pltpu.async_copy(
                table_hbm.at[idx_v.at[i]], rows_v.at[i], gsem)
            for i in range(RCHUNK)
        ]
        for g in gds:
            g.wait()
        pltpu.sync_copy(rows_v, out_hbm.at[pl.ds(r0, RCHUNK)])
        return carry

    lax.fori_loop(0, N_CHUNKS, body, 0)


def kernel(coin_id, table):
    mesh = plsc.VectorSubcoreMesh(core_axis_name="c", subcore_axis_name="s")
    k = pl.kernel(
        _emb_body,
        mesh=mesh,
        out_type=jax.ShapeDtypeStruct((BATCH, HIST, EMBED_DIM), jnp.float32),
        scratch_types=(
            [pltpu.VMEM((RCHUNK, HIST), jnp.int32)]
            + [pltpu.VMEM((RCHUNK, HIST, EMBED_DIM), jnp.float32)]
            + [pltpu.SemaphoreType.DMA]
        ),
        compiler_params=pltpu.CompilerParams(use_tc_tiling_on_sc=False),
    )
    return k(coin_id, table)


# speedup vs baseline: 2.4166x; 1.7178x over previous
import jax
import jax.numpy as jnp
from jax import lax
from jax.experimental import pallas as pl
from jax.experimental.pallas import tpu as pltpu
from jax.experimental.pallas import tpu_sc as plsc

N_COINS = 100000
EMBED_DIM = 16
BATCH = 16384
HIST = 50

NC = 2
NS = 16
H_PER_CORE = HIST // NC
CHUNK = 4096
N_CHUNKS = BATCH // CHUNK
LANES = 16
VSTEPS = CHUNK // LANES


def _emb_body(idx_hbm, tab_hbm, out_hbm, tabrow_v, idx_v, res_v):
    e = lax.axis_index("s")
    core = lax.axis_index("c")

    pltpu.sync_copy(tab_hbm.at[e], tabrow_v)

    def h_body(hh, carry):
        h = core * H_PER_CORE + hh

        def c_body(ci, carry2):
            c0 = pl.multiple_of(ci * CHUNK, 8)
            pltpu.sync_copy(idx_hbm.at[h, pl.ds(c0, CHUNK)], idx_v)

            def v_body(k, carry3):
                o = pl.multiple_of(k * LANES, 8)
                iv = idx_v[pl.ds(o, LANES)]
                res_v[pl.ds(o, LANES)] = plsc.load_gather(tabrow_v, [iv])
                return carry3

            lax.fori_loop(0, VSTEPS, v_body, 0)
            pltpu.sync_copy(res_v, out_hbm.at[h, e, pl.ds(c0, CHUNK)])
            return carry2

        lax.fori_loop(0, N_CHUNKS, c_body, 0)
        return carry

    lax.fori_loop(0, H_PER_CORE, h_body, 0)


def kernel(coin_id, table):
    mesh = plsc.VectorSubcoreMesh(core_axis_name="c", subcore_axis_name="s")
    k = pl.kernel(
        _emb_body,
        mesh=mesh,
        out_type=jax.ShapeDtypeStruct((HIST, EMBED_DIM, BATCH), jnp.float32),
        scratch_types=(
            [pltpu.VMEM((N_COINS,), jnp.float32)]
            + [pltpu.VMEM((CHUNK,), jnp.int32)]
            + [pltpu.VMEM((CHUNK,), jnp.float32)]
        ),
        compiler_params=pltpu.CompilerParams(
            use_tc_tiling_on_sc=False, needs_layout_passes=False),
    )
    out_t = k(coin_id.T, table.T)
    return out_t.transpose(2, 0, 1)


# speedup vs baseline: 3.3088x; 1.3692x over previous
import jax
import jax.numpy as jnp
from jax import lax
from jax.experimental import pallas as pl
from jax.experimental.pallas import tpu as pltpu
from jax.experimental.pallas import tpu_sc as plsc

N_COINS = 100000
EMBED_DIM = 16
BATCH = 16384
HIST = 50

NC = 2
NS = 16
H_PER_CORE = HIST // NC
CHUNK = 4096
N_CHUNKS = BATCH // CHUNK
N_PAIRS = H_PER_CORE * N_CHUNKS
LANES = 16
UNROLL = 8
VSTEPS = CHUNK // (LANES * UNROLL)


def _emb_body(idx_hbm, tab_hbm, out_hbm, tabrow_v,
              idx0, idx1, res0, res1, isem0, isem1, ssem0, ssem1):
    e = lax.axis_index("s")
    core = lax.axis_index("c")
    h_base = core * H_PER_CORE

    pltpu.sync_copy(tab_hbm.at[e], tabrow_v)

    idxs = (idx0, idx1)
    ress = (res0, res1)
    isems = (isem0, isem1)
    ssems = (ssem0, ssem1)

    def locate(p):
        h = h_base + lax.shift_right_logical(p, 2)
        c0 = pl.multiple_of(lax.shift_left(lax.bitwise_and(p, 3), 12), 8)
        return h, c0

    def idx_load(p, b):
        h, c0 = locate(p)
        return pltpu.async_copy(
            idx_hbm.at[h, pl.ds(c0, CHUNK)], idxs[b], isems[b])

    def compute(b):
        def v_body(k, carry):
            base = pl.multiple_of(k * (LANES * UNROLL), 8)
            for i in range(UNROLL):
                o = base + i * LANES
                iv = idxs[b][pl.ds(o, LANES)]
                ress[b][pl.ds(o, LANES)] = plsc.load_gather(tabrow_v, [iv])
            return carry

        lax.fori_loop(0, VSTEPS, v_body, 0)

    def store(p, b):
        h, c0 = locate(p)
        return pltpu.async_copy(
            ress[b], out_hbm.at[h, e, pl.ds(c0, CHUNK)], ssems[b])

    def g_body(g, carry):
        pA = g * 2
        pB = pA + 1
        dA = idx_load(pA, 0)
        dB = idx_load(pB, 1)
        dA.wait()
        compute(0)
        sA = store(pA, 0)
        dB.wait()
        compute(1)
        sB = store(pB, 1)
        sA.wait()
        sB.wait()
        return carry

    lax.fori_loop(0, N_PAIRS // 2, g_body, 0)


def kernel(coin_id, table):
    mesh = plsc.VectorSubcoreMesh(core_axis_name="c", subcore_axis_name="s")
    k = pl.kernel(
        _emb_body,
        mesh=mesh,
        out_type=jax.ShapeDtypeStruct((HIST, EMBED_DIM, BATCH), jnp.float32),
        scratch_types=(
            [pltpu.VMEM((N_COINS,), jnp.float32)]
            + [pltpu.VMEM((CHUNK,), jnp.int32)] * 2
            + [pltpu.VMEM((CHUNK,), jnp.float32)] * 2
            + [pltpu.SemaphoreType.DMA] * 4
        ),
        compiler_params=pltpu.CompilerParams(
            use_tc_tiling_on_sc=False, needs_layout_passes=False),
    )
    out_t = k(coin_id.T, table.T)
    return out_t.transpose(2, 0, 1)


# speedup vs baseline: 3.3298x; 1.0063x over previous
import jax
import jax.numpy as jnp
from jax import lax
from jax.experimental import pallas as pl
from jax.experimental.pallas import tpu as pltpu
from jax.experimental.pallas import tpu_sc as plsc

N_COINS = 100000
EMBED_DIM = 16
BATCH = 16384
HIST = 50

NC = 2
NS = 16
H_PER_CORE = HIST // NC
CHUNK = 4096
N_CHUNKS = BATCH // CHUNK
N_PAIRS = H_PER_CORE * N_CHUNKS
LANES = 16
UNROLL = 16
VSTEPS = CHUNK // (LANES * UNROLL)


def _emb_body(idx_hbm, tab_hbm, out_hbm, tabrow_v,
              idx0, idx1, res0, res1, isem0, isem1, ssem0, ssem1):
    e = lax.axis_index("s")
    core = lax.axis_index("c")
    h_base = core * H_PER_CORE

    pltpu.sync_copy(tab_hbm.at[e], tabrow_v)

    idxs = (idx0, idx1)
    ress = (res0, res1)
    isems = (isem0, isem1)
    ssems = (ssem0, ssem1)

    def locate(p):
        h = h_base + lax.shift_right_logical(p, 2)
        c0 = pl.multiple_of(lax.shift_left(lax.bitwise_and(p, 3), 12), 8)
        return h, c0

    def idx_load(p, b):
        h, c0 = locate(p)
        return pltpu.async_copy(
            idx_hbm.at[h, pl.ds(c0, CHUNK)], idxs[b], isems[b])

    def compute(b):
        def v_body(k, carry):
            base = pl.multiple_of(k * (LANES * UNROLL), 8)
            for i in range(UNROLL):
                o = base + i * LANES
                iv = idxs[b][pl.ds(o, LANES)]
                ress[b][pl.ds(o, LANES)] = plsc.load_gather(tabrow_v, [iv])
            return carry

        lax.fori_loop(0, VSTEPS, v_body, 0)

    def store_desc(p, b):
        h, c0 = locate(p)
        return pltpu.make_async_copy(
            ress[b], out_hbm.at[h, e, pl.ds(c0, CHUNK)], ssems[b])

    def g_body(g, carry):
        pA = g * 2
        pB = pA + 1

        @pl.when(g > 0)
        def _():
            store_desc(pA - 2, 0).wait()
            store_desc(pB - 2, 1).wait()

        dA = idx_load(pA, 0)
        dB = idx_load(pB, 1)
        dA.wait()
        compute(0)
        store_desc(pA, 0).start()
        dB.wait()
        compute(1)
        store_desc(pB, 1).start()
        return carry

    lax.fori_loop(0, N_PAIRS // 2, g_body, 0)
    store_desc(N_PAIRS - 2, 0).wait()
    store_desc(N_PAIRS - 1, 1).wait()


def kernel(coin_id, table):
    mesh = plsc.VectorSubcoreMesh(core_axis_name="c", subcore_axis_name="s")
    k = pl.kernel(
        _emb_body,
        mesh=mesh,
        out_type=jax.ShapeDtypeStruct((HIST, EMBED_DIM, BATCH), jnp.float32),
        scratch_types=(
            [pltpu.VMEM((N_COINS,), jnp.float32)]
            + [pltpu.VMEM((CHUNK,), jnp.int32)] * 2
            + [pltpu.VMEM((CHUNK,), jnp.float32)] * 2
            + [pltpu.SemaphoreType.DMA] * 4
        ),
        compiler_params=pltpu.CompilerParams(
            use_tc_tiling_on_sc=False, needs_layout_passes=False),
    )
    out_t = k(coin_id.T, table.T)
    return out_t.transpose(2, 0, 1)


# speedup vs baseline: 4.2142x; 1.2656x over previous
import jax
import jax.numpy as jnp
from jax import lax
from jax.experimental import pallas as pl
from jax.experimental.pallas import tpu as pltpu
from jax.experimental.pallas import tpu_sc as plsc

N_COINS = 100000
EMBED_DIM = 16
BATCH = 16384
HIST = 50

NC = 2
NS = 16
H_PER_CORE = HIST // NC
CHUNK = 4096
N_CHUNKS = BATCH // CHUNK
N_PAIRS = H_PER_CORE * N_CHUNKS
LANES = 16
UNROLL = 16
VSTEPS = CHUNK // (LANES * UNROLL)


def _emb_body(idx_hbm, tab_hbm, out_hbm, tabrow_v,
              idx0, idx1, res0, res1, isem0, isem1, ssem0, ssem1):
    e = lax.axis_index("s")
    core = lax.axis_index("c")
    h_base = core * H_PER_CORE

    pltpu.sync_copy(tab_hbm.at[e], tabrow_v)

    idxs = (idx0, idx1)
    ress = (res0, res1)
    isems = (isem0, isem1)
    ssems = (ssem0, ssem1)

    def locate(p):
        h = h_base + lax.shift_right_logical(p, 2)
        c0 = pl.multiple_of(lax.shift_left(lax.bitwise_and(p, 3), 12), 8)
        return h, c0

    def idx_load(p, b):
        h, c0 = locate(p)
        return pltpu.async_copy(
            idx_hbm.at[h, pl.ds(c0, CHUNK)], idxs[b], isems[b])

    ehi = lax.shift_right_logical(e, 3)
    elo = lax.bitwise_and(e, 7)

    def compute(b):
        def v_body(k, carry):
            base = pl.multiple_of(k * (LANES * UNROLL), 8)
            for i in range(UNROLL):
                o = base + i * LANES
                iv = idxs[b][pl.ds(o, LANES)]
                row = 2 * k + i // 8
                col = (i % 8) * LANES
                ress[b][row, pl.ds(col, LANES)] = plsc.load_gather(
                    tabrow_v, [iv])
            return carry

        lax.fori_loop(0, VSTEPS, v_body, 0)

    def store_desc(p, b):
        h, c0 = locate(p)
        bt0 = lax.shift_right_logical(c0, 7)
        return pltpu.make_async_copy(
            ress[b], out_hbm.at[h, ehi, pl.ds(bt0, CHUNK // 128), elo],
            ssems[b])

    def g_body(g, carry):
        pA = g * 2
        pB = pA + 1

        @pl.when(g > 0)
        def _():
            store_desc(pA - 2, 0).wait()
            store_desc(pB - 2, 1).wait()

        dA = idx_load(pA, 0)
        dB = idx_load(pB, 1)
        dA.wait()
        compute(0)
        store_desc(pA, 0).start()
        dB.wait()
        compute(1)
        store_desc(pB, 1).start()
        return carry

    lax.fori_loop(0, N_PAIRS // 2, g_body, 0)
    store_desc(N_PAIRS - 2, 0).wait()
    store_desc(N_PAIRS - 1, 1).wait()


def kernel(coin_id, table):
    mesh = plsc.VectorSubcoreMesh(core_axis_name="c", subcore_axis_name="s")
    k = pl.kernel(
        _emb_body,
        mesh=mesh,
        out_type=jax.ShapeDtypeStruct(
            (HIST, EMBED_DIM // 8, BATCH // 128, 8, 128), jnp.float32),
        scratch_types=(
            [pltpu.VMEM((N_COINS,), jnp.float32)]
            + [pltpu.VMEM((CHUNK,), jnp.int32)] * 2
            + [pltpu.VMEM((CHUNK // 128, 128), jnp.float32)] * 2
            + [pltpu.SemaphoreType.DMA] * 4
        ),
        compiler_params=pltpu.CompilerParams(
            use_tc_tiling_on_sc=False, needs_layout_passes=False),
    )
    out_t = k(coin_id.T, table.T)
    return out_t.transpose(2, 4, 0, 1, 3).reshape(BATCH, HIST, EMBED_DIM)


# speedup vs baseline: 5.3352x; 1.2660x over previous
import jax
import jax.numpy as jnp
from jax import lax
from jax.experimental import pallas as pl
from jax.experimental.pallas import tpu as pltpu
from jax.experimental.pallas import tpu_sc as plsc

N_COINS = 100000
EMBED_DIM = 16
BATCH = 16384
HIST = 50

NC = 2
NS = 16
H_PER_CORE = HIST // NC
CHUNK = 4096
N_CHUNKS = BATCH // CHUNK
N_PAIRS = H_PER_CORE * N_CHUNKS
LANES = 16
UNROLL = 16
VSTEPS = CHUNK // (LANES * UNROLL)


def _emb_body(idx_hbm, tab_hbm, out_hbm, tabrow_v,
              idx0, idx1, res0, res1, isem0, isem1, ssem0, ssem1):
    e = lax.axis_index("s")
    core = lax.axis_index("c")
    h_base = core * H_PER_CORE

    pltpu.sync_copy(tab_hbm.at[e], tabrow_v)

    idxs = (idx0, idx1)
    ress = (res0, res1)
    isems = (isem0, isem1)
    ssems = (ssem0, ssem1)

    def locate(p):
        h = h_base + lax.shift_right_logical(p, 2)
        c0 = pl.multiple_of(lax.shift_left(lax.bitwise_and(p, 3), 12), 8)
        return h, c0

    def idx_desc(p, b):
        h, c0 = locate(p)
        return pltpu.make_async_copy(
            idx_hbm.at[h, pl.ds(c0, CHUNK)], idxs[b], isems[b])

    ehi = lax.shift_right_logical(e, 3)
    elo = lax.bitwise_and(e, 7)

    def compute(b):
        def v_body(k, carry):
            base = pl.multiple_of(k * (LANES * UNROLL), 8)
            for i in range(UNROLL):
                o = base + i * LANES
                iv = idxs[b][pl.ds(o, LANES)]
                row = 2 * k + i // 8
                col = (i % 8) * LANES
                ress[b][row, pl.ds(col, LANES)] = plsc.load_gather(
                    tabrow_v, [iv])
            return carry

        lax.fori_loop(0, VSTEPS, v_body, 0)

    def store_desc(p, b):
        h, c0 = locate(p)
        bt0 = lax.shift_right_logical(c0, 7)
        return pltpu.make_async_copy(
            ress[b], out_hbm.at[h, ehi, pl.ds(bt0, CHUNK // 128), elo],
            ssems[b])

    idx_desc(0, 0).start()
    idx_desc(1, 1).start()

    def g_body(g, carry):
        pA = g * 2
        pB = pA + 1

        @pl.when(g > 0)
        def _():
            store_desc(pA - 2, 0).wait()
            store_desc(pB - 2, 1).wait()

        idx_desc(pA, 0).wait()
        compute(0)

        @pl.when(pA + 2 < N_PAIRS)
        def _():
            idx_desc(pA + 2, 0).start()

        store_desc(pA, 0).start()
        idx_desc(pB, 1).wait()
        compute(1)

        @pl.when(pB + 2 < N_PAIRS)
        def _():
            idx_desc(pB + 2, 1).start()

        store_desc(pB, 1).start()
        return carry

    lax.fori_loop(0, N_PAIRS // 2, g_body, 0)
    store_desc(N_PAIRS - 2, 0).wait()
    store_desc(N_PAIRS - 1, 1).wait()


def kernel(coin_id, table):
    mesh = plsc.VectorSubcoreMesh(core_axis_name="c", subcore_axis_name="s")
    k = pl.kernel(
        _emb_body,
        mesh=mesh,
        out_type=jax.ShapeDtypeStruct(
            (HIST, EMBED_DIM // 8, BATCH // 128, 8, 128), jnp.float32),
        scratch_types=(
            [pltpu.VMEM((N_COINS,), jnp.float32)]
            + [pltpu.VMEM((CHUNK,), jnp.int32)] * 2
            + [pltpu.VMEM((CHUNK // 128, 128), jnp.float32)] * 2
            + [pltpu.SemaphoreType.DMA] * 4
        ),
        compiler_params=pltpu.CompilerParams(
            use_tc_tiling_on_sc=False, needs_layout_passes=False),
    )
    out_t = k(coin_id.T, table.T)
    return out_t.transpose(2, 4, 0, 1, 3).reshape(BATCH, HIST, EMBED_DIM)
